# Initial kernel scaffold; baseline (speedup 1.0000x reference)
import jax
import jax.numpy as jnp
from jax.experimental import pallas as pl


def kernel(xyz1, xyz2, points1, points2, W1, b1, g1, be1, W2, b2, g2, be2):
    raise NotImplementedError("write your pallas kernel here")


# speedup vs baseline: 22.0954x; 22.0954x over previous
import functools

import jax
import jax.numpy as jnp
from jax import lax
from jax.experimental import pallas as pl
from jax.experimental.pallas import tpu as pltpu

EPS_BN = 1e-5
EPS_W = 1e-8


def _lane_fold(x, width=128):
    acc = x[:, :width]
    for i in range(1, x.shape[1] // width):
        acc = acc + x[:, i * width:(i + 1) * width]
    return acc


def _stage_a_kernel(x1_ref, x2_ref, p1_ref, p2_ref, w1a_ref, w1b_ref, b1_ref,
                    y1_ref, stats_ref, p2w_ref, *, bn, S):
    b = pl.program_id(0)
    nb = pl.program_id(1)

    @pl.when(nb == 0)
    def _():
        p2w_ref[...] = jnp.dot(w1b_ref[...], p2_ref[0],
                               preferred_element_type=jnp.float32)

    x1 = x1_ref[0]
    x2t = x2_ref[0]
    cross = jnp.dot(x2t[:, :8], x1,
                    preferred_element_type=jnp.float32)
    n2 = jnp.sum(x2t * x2t, axis=1, keepdims=True)
    n1 = jnp.sum(x1 * x1, axis=0, keepdims=True)
    d = ((-2.0) * cross + n1) + n2

    iota = lax.broadcasted_iota(jnp.int32, (S, bn), 0)
    recips = []
    masks = []
    for _ in range(3):
        m = jnp.min(d, axis=0, keepdims=True)
        c = jnp.min(jnp.where(d == m, iota, S), axis=0, keepdims=True)
        mask = iota == c
        masks.append(mask)
        recips.append(1.0 / (m + EPS_W))
        d = jnp.where(mask, jnp.float32(jnp.inf), d)
    norm = recips[0] + recips[1] + recips[2]
    at = (jnp.where(masks[0], recips[0], 0.0)
          + jnp.where(masks[1], recips[1], 0.0)
          + jnp.where(masks[2], recips[2], 0.0)) / norm

    y = (jnp.dot(w1a_ref[...], p1_ref[0], preferred_element_type=jnp.float32)
         + jnp.dot(p2w_ref[...], at, preferred_element_type=jnp.float32)
         + b1_ref[...])
    y1_ref[0] = y

    @pl.when(jnp.logical_and(b == 0, nb == 0))
    def _():
        stats_ref[...] = jnp.zeros_like(stats_ref)
    stats_ref[0, :, :] = stats_ref[0, :, :] + _lane_fold(y)
    stats_ref[1, :, :] = stats_ref[1, :, :] + _lane_fold(y * y)


def _stage_b_kernel(y1_ref, w2_ref, a1_ref, c1_ref, b2_ref,
                    y2_ref, stats_ref, *, bn):
    b = pl.program_id(0)
    nb = pl.program_id(1)
    x = jnp.maximum(a1_ref[...] * y1_ref[0] + c1_ref[...], 0.0)
    y = (jnp.dot(w2_ref[...], x, preferred_element_type=jnp.float32)
         + b2_ref[...])
    y2_ref[0] = y
    @pl.when(jnp.logical_and(b == 0, nb == 0))
    def _():
        stats_ref[...] = jnp.zeros_like(stats_ref)
    stats_ref[0, :, :] = stats_ref[0, :, :] + _lane_fold(y)
    stats_ref[1, :, :] = stats_ref[1, :, :] + _lane_fold(y * y)


def _stage_c_kernel(y2_ref, a2_ref, c2_ref, o_ref):
    o_ref[0] = jnp.maximum(a2_ref[...] * y2_ref[0] + c2_ref[...], 0.0)


def _bn_coeffs(stats, g, be, count):
    s = jnp.sum(stats, axis=-1)
    mean = s[0] / count
    var = s[1] / count - mean * mean
    a = g / jnp.sqrt(var + EPS_BN)
    c = be - mean * a
    return a[:, None], c[:, None]


def kernel(xyz1, xyz2, points1, points2, W1, b1, g1, be1, W2, b2, g2, be2):
    B, _, N = xyz1.shape
    S = xyz2.shape[2]
    C1 = points1.shape[1]
    C2 = points2.shape[1]
    Co1 = W1.shape[0]
    Co2 = W2.shape[0]
    f32 = jnp.float32

    bn = 256
    nb = N // bn

    pad = [(0, 0), (0, 5), (0, 0)]
    xyz1p = jnp.pad(xyz1, pad)
    xyz2p = jnp.pad(jnp.transpose(xyz2, (0, 2, 1)), [(0, 0), (0, 0), (0, 125)])
    W1a = W1[:, :C1]
    W1b = W1[:, C1:]
    b1c = b1[:, None]
    b2c = b2[:, None]

    grid = (B, nb)
    y1, stats1 = pl.pallas_call(
        functools.partial(_stage_a_kernel, bn=bn, S=S),
        grid=grid,
        in_specs=[
            pl.BlockSpec((1, 8, bn), lambda b, n: (b, 0, n)),
            pl.BlockSpec((1, S, 128), lambda b, n: (b, 0, 0)),
            pl.BlockSpec((1, C1, bn), lambda b, n: (b, 0, n)),
            pl.BlockSpec((1, C2, S), lambda b, n: (b, 0, 0)),
            pl.BlockSpec((Co1, C1), lambda b, n: (0, 0)),
            pl.BlockSpec((Co1, C2), lambda b, n: (0, 0)),
            pl.BlockSpec((Co1, 1), lambda b, n: (0, 0)),
        ],
        out_specs=[
            pl.BlockSpec((1, Co1, bn), lambda b, n: (b, 0, n)),
            pl.BlockSpec((2, Co1, 128), lambda b, n: (0, 0, 0)),
        ],
        out_shape=[
            jax.ShapeDtypeStruct((B, Co1, N), f32),
            jax.ShapeDtypeStruct((2, Co1, 128), f32),
        ],
        scratch_shapes=[pltpu.VMEM((Co1, S), f32)],
        compiler_params=pltpu.CompilerParams(
            dimension_semantics=("arbitrary", "arbitrary")),
    )(xyz1p, xyz2p, points1, points2, W1a, W1b, b1c)

    a1, c1 = _bn_coeffs(stats1, g1, be1, B * N)

    y2, stats2 = pl.pallas_call(
        functools.partial(_stage_b_kernel, bn=bn),
        grid=grid,
        in_specs=[
            pl.BlockSpec((1, Co1, bn), lambda b, n: (b, 0, n)),
            pl.BlockSpec((Co2, Co1), lambda b, n: (0, 0)),
            pl.BlockSpec((Co1, 1), lambda b, n: (0, 0)),
            pl.BlockSpec((Co1, 1), lambda b, n: (0, 0)),
            pl.BlockSpec((Co2, 1), lambda b, n: (0, 0)),
        ],
        out_specs=[
            pl.BlockSpec((1, Co2, bn), lambda b, n: (b, 0, n)),
            pl.BlockSpec((2, Co2, 128), lambda b, n: (0, 0, 0)),
        ],
        out_shape=[
            jax.ShapeDtypeStruct((B, Co2, N), f32),
            jax.ShapeDtypeStruct((2, Co2, 128), f32),
        ],
        compiler_params=pltpu.CompilerParams(
            dimension_semantics=("arbitrary", "arbitrary")),
    )(y1, W2, a1, c1, b2c)

    a2, c2 = _bn_coeffs(stats2, g2, be2, B * N)

    bnc = min(1024, N)
    out = pl.pallas_call(
        _stage_c_kernel,
        grid=(B, N // bnc),
        in_specs=[
            pl.BlockSpec((1, Co2, bnc), lambda b, n: (b, 0, n)),
            pl.BlockSpec((Co2, 1), lambda b, n: (0, 0)),
            pl.BlockSpec((Co2, 1), lambda b, n: (0, 0)),
        ],
        out_specs=pl.BlockSpec((1, Co2, bnc), lambda b, n: (b, 0, n)),
        out_shape=jax.ShapeDtypeStruct((B, Co2, N), f32),
        compiler_params=pltpu.CompilerParams(
            dimension_semantics=("arbitrary", "arbitrary")),
    )(y2, a2, c2)

    return out


# speedup vs baseline: 26.1853x; 1.1851x over previous
import functools

import jax
import jax.numpy as jnp
from jax import lax
from jax.experimental import pallas as pl
from jax.experimental.pallas import tpu as pltpu

EPS_BN = 1e-5
EPS_W = 1e-8


def _lane_fold(x, width=128):
    acc = x[:, :width]
    for i in range(1, x.shape[1] // width):
        acc = acc + x[:, i * width:(i + 1) * width]
    return acc


def _stage_a_kernel(x1_ref, x2_ref, p1_ref, p2_ref, w1a_ref, w1b_ref, b1_ref,
                    y1_ref, stats_ref, p2w_ref, acc_ref, *, bn, S, nb_last):
    b = pl.program_id(0)
    nb = pl.program_id(1)

    @pl.when(nb == 0)
    def _():
        p2w_ref[...] = jnp.dot(w1b_ref[...], p2_ref[0],
                               preferred_element_type=jnp.float32)

    x1 = x1_ref[0]
    x2t = x2_ref[0]
    cross = jnp.dot(x2t[:, :8], x1,
                    preferred_element_type=jnp.float32)
    n2 = jnp.sum(x2t * x2t, axis=1, keepdims=True)
    n1 = jnp.sum(x1 * x1, axis=0, keepdims=True)
    d = ((-2.0) * cross + n1) + n2

    d0 = d
    recips = []
    mins = []
    for _ in range(3):
        m = jnp.min(d, axis=0, keepdims=True)
        mins.append(m)
        recips.append(1.0 / (m + EPS_W))
        d = jnp.where(d == m, jnp.float32(jnp.inf), d)
    norm = recips[0] + recips[1] + recips[2]
    at = (jnp.where(d0 == mins[0], recips[0], 0.0)
          + jnp.where(d0 == mins[1], recips[1], 0.0)
          + jnp.where(d0 == mins[2], recips[2], 0.0)) / norm

    y = (jnp.dot(w1a_ref[...], p1_ref[0], preferred_element_type=jnp.float32)
         + jnp.dot(p2w_ref[...], at, preferred_element_type=jnp.float32)
         + b1_ref[...])
    y1_ref[0] = y
    _accumulate_stats(y, stats_ref, acc_ref, b, nb, nb_last)


def _accumulate_stats(y, stats_ref, acc_ref, b, nb, nb_last):
    @pl.when(jnp.logical_and(b == 0, nb == 0))
    def _():
        acc_ref[...] = jnp.zeros_like(acc_ref)
    acc_ref[0, :, :] = acc_ref[0, :, :] + _lane_fold(y)
    acc_ref[1, :, :] = acc_ref[1, :, :] + _lane_fold(y * y)

    @pl.when(jnp.logical_and(b == pl.num_programs(0) - 1, nb == nb_last))
    def _():
        stats_ref[...] = acc_ref[...]


def _stage_b_kernel(y1_ref, w2_ref, a1_ref, c1_ref, b2_ref,
                    y2_ref, stats_ref, acc_ref, *, bn, nb_last):
    b = pl.program_id(0)
    nb = pl.program_id(1)
    x = jnp.maximum(a1_ref[...] * y1_ref[0] + c1_ref[...], 0.0)
    y = (jnp.dot(w2_ref[...], x, preferred_element_type=jnp.float32)
         + b2_ref[...])
    y2_ref[0] = y
    _accumulate_stats(y, stats_ref, acc_ref, b, nb, nb_last)


def _stage_c_kernel(y2_ref, a2_ref, c2_ref, o_ref):
    o_ref[0] = jnp.maximum(a2_ref[...] * y2_ref[0] + c2_ref[...], 0.0)


def _bn_coeffs(stats, g, be, count):
    s = jnp.sum(stats, axis=-1)
    mean = s[0] / count
    var = s[1] / count - mean * mean
    a = g / jnp.sqrt(var + EPS_BN)
    c = be - mean * a
    return a[:, None], c[:, None]


def kernel(xyz1, xyz2, points1, points2, W1, b1, g1, be1, W2, b2, g2, be2):
    B, _, N = xyz1.shape
    S = xyz2.shape[2]
    C1 = points1.shape[1]
    C2 = points2.shape[1]
    Co1 = W1.shape[0]
    Co2 = W2.shape[0]
    f32 = jnp.float32

    bn = 256
    nb = N // bn

    pad = [(0, 0), (0, 5), (0, 0)]
    xyz1p = jnp.pad(xyz1, pad)
    xyz2p = jnp.pad(jnp.transpose(xyz2, (0, 2, 1)), [(0, 0), (0, 0), (0, 125)])
    W1a = W1[:, :C1]
    W1b = W1[:, C1:]
    b1c = b1[:, None]
    b2c = b2[:, None]

    grid = (B, nb)
    y1, stats1 = pl.pallas_call(
        functools.partial(_stage_a_kernel, bn=bn, S=S, nb_last=nb - 1),
        grid=grid,
        in_specs=[
            pl.BlockSpec((1, 8, bn), lambda b, n: (b, 0, n)),
            pl.BlockSpec((1, S, 128), lambda b, n: (b, 0, 0)),
            pl.BlockSpec((1, C1, bn), lambda b, n: (b, 0, n)),
            pl.BlockSpec((1, C2, S), lambda b, n: (b, 0, 0)),
            pl.BlockSpec((Co1, C1), lambda b, n: (0, 0)),
            pl.BlockSpec((Co1, C2), lambda b, n: (0, 0)),
            pl.BlockSpec((Co1, 1), lambda b, n: (0, 0)),
        ],
        out_specs=[
            pl.BlockSpec((1, Co1, bn), lambda b, n: (b, 0, n)),
            pl.BlockSpec((2, Co1, 128), lambda b, n: (0, 0, 0)),
        ],
        out_shape=[
            jax.ShapeDtypeStruct((B, Co1, N), f32),
            jax.ShapeDtypeStruct((2, Co1, 128), f32),
        ],
        scratch_shapes=[pltpu.VMEM((Co1, S), f32),
                        pltpu.VMEM((2, Co1, 128), f32)],
        compiler_params=pltpu.CompilerParams(
            dimension_semantics=("arbitrary", "arbitrary")),
    )(xyz1p, xyz2p, points1, points2, W1a, W1b, b1c)

    a1, c1 = _bn_coeffs(stats1, g1, be1, B * N)

    y2, stats2 = pl.pallas_call(
        functools.partial(_stage_b_kernel, bn=bn, nb_last=nb - 1),
        grid=grid,
        in_specs=[
            pl.BlockSpec((1, Co1, bn), lambda b, n: (b, 0, n)),
            pl.BlockSpec((Co2, Co1), lambda b, n: (0, 0)),
            pl.BlockSpec((Co1, 1), lambda b, n: (0, 0)),
            pl.BlockSpec((Co1, 1), lambda b, n: (0, 0)),
            pl.BlockSpec((Co2, 1), lambda b, n: (0, 0)),
        ],
        out_specs=[
            pl.BlockSpec((1, Co2, bn), lambda b, n: (b, 0, n)),
            pl.BlockSpec((2, Co2, 128), lambda b, n: (0, 0, 0)),
        ],
        out_shape=[
            jax.ShapeDtypeStruct((B, Co2, N), f32),
            jax.ShapeDtypeStruct((2, Co2, 128), f32),
        ],
        scratch_shapes=[pltpu.VMEM((2, Co2, 128), f32)],
        compiler_params=pltpu.CompilerParams(
            dimension_semantics=("arbitrary", "arbitrary")),
    )(y1, W2, a1, c1, b2c)

    a2, c2 = _bn_coeffs(stats2, g2, be2, B * N)

    bnc = min(1024, N)
    out = pl.pallas_call(
        _stage_c_kernel,
        grid=(B, N // bnc),
        in_specs=[
            pl.BlockSpec((1, Co2, bnc), lambda b, n: (b, 0, n)),
            pl.BlockSpec((Co2, 1), lambda b, n: (0, 0)),
            pl.BlockSpec((Co2, 1), lambda b, n: (0, 0)),
        ],
        out_specs=pl.BlockSpec((1, Co2, bnc), lambda b, n: (b, 0, n)),
        out_shape=jax.ShapeDtypeStruct((B, Co2, N), f32),
        compiler_params=pltpu.CompilerParams(
            dimension_semantics=("arbitrary", "arbitrary")),
    )(y2, a2, c2)

    return out
